# baseline (speedup 1.0000x reference)
import jax
import jax.numpy as jnp
from jax import lax
from jax.experimental import pallas as pl
from jax.experimental.pallas import tpu as pltpu
from jax.experimental.pallas import tpu_sc as plsc

_N_TILES = 1024
_EMBED = 32
_HID = 256
_NA = 4
_B = 4096

_NC, _NS = 2, 16
_NW = _NC * _NS
_NCHUNK = 4
_CB = _B // _NCHUNK
_BPW = _CB // _NW


def _sc_gather_body(table_hbm, board_hbm, out_hbm, table_v, idx_v, emb_v,
                    sem_out, sem_idx):
    w = lax.axis_index("s") * _NC + lax.axis_index("c")
    base = w * _BPW
    pltpu.sync_copy(table_hbm, table_v)
    iota = lax.iota(jnp.int32, 16)
    c127 = jnp.full((16,), 127, jnp.int32)

    pltpu.async_copy(board_hbm.at[base], idx_v.at[0], sem_idx).wait()

    def row_body(i, carry):
        br = base + i
        p = lax.rem(i, 2)
        @pl.when(i + 1 < _BPW)
        def _():
            pltpu.async_copy(board_hbm.at[br + 1], idx_v.at[1 - p], sem_idx)
        @pl.when(i >= 2)
        def _():
            pltpu.make_async_copy(emb_v.at[p], out_hbm.at[base], sem_out).wait()

        @plsc.parallel_loop(0, _N_TILES // 4, unroll=4)
        def grp_body(g):
            n0 = g * 4
            rowv = jnp.full((16,), n0 >> 7, jnp.int32)
            col0 = jnp.full((16,), n0 & 127, jnp.int32)
            for c in range(4):
                vsp = plsc.load_gather(idx_v.at[p], [rowv, col0 + c])
                srow = vsp >> 2
                e0 = ((vsp << 5) & c127) + iota
                g0 = plsc.load_gather(table_v, [srow, e0])
                g1 = plsc.load_gather(table_v, [srow, e0 + 16])
                emb_v[p, g, pl.ds(c * 32, 16)] = g0
                emb_v[p, g, pl.ds(c * 32 + 16, 16)] = g1
        pltpu.async_copy(emb_v.at[p], out_hbm.at[br], sem_out)

        @pl.when(i + 1 < _BPW)
        def _():
            pltpu.make_async_copy(board_hbm.at[br], idx_v.at[1 - p],
                                  sem_idx).wait()
        return carry

    lax.fori_loop(0, _BPW, row_body, 0)
    pltpu.make_async_copy(emb_v.at[0], out_hbm.at[base], sem_out).wait()
    pltpu.make_async_copy(emb_v.at[0], out_hbm.at[base], sem_out).wait()


def _sc_gather(emb_table2, board3):
    mesh = plsc.VectorSubcoreMesh(core_axis_name="c", subcore_axis_name="s")
    f = pl.kernel(
        _sc_gather_body,
        out_type=jax.ShapeDtypeStruct((_CB, _N_TILES * _EMBED // 128, 128),
                                      jnp.float32),
        mesh=mesh,
        scratch_types=[
            pltpu.VMEM((_N_TILES * _EMBED // 128, 128), jnp.float32),
            pltpu.VMEM((2, 8, 128), jnp.int32),
            pltpu.VMEM((2, _N_TILES * _EMBED // 128, 128), jnp.float32),
            pltpu.SemaphoreType.DMA,
            pltpu.SemaphoreType.DMA,
        ],
        compiler_params=pltpu.CompilerParams(use_tc_tiling_on_sc=True,
                                             needs_layout_passes=False),
    )
    return f(emb_table2, board3)


_BB = 512
_KB = 4096
_K = _N_TILES * _EMBED


def _mlp_body(flat_ref, w1_ref, b1_ref, w2_ref, b2_ref, w3_ref, b3_ref,
              out_ref, acc_ref):
    k = pl.program_id(0)
    b = pl.program_id(1)
    nk = pl.num_programs(0)
    x = flat_ref[...].reshape(_BB, _KB)
    part = jnp.dot(x, w1_ref[...], preferred_element_type=jnp.float32)
    sl = pl.ds(b * _BB, _BB)

    @pl.when(k == 0)
    def _():
        acc_ref[sl, :] = part

    @pl.when(k > 0)
    def _():
        acc_ref[sl, :] = acc_ref[sl, :] + part

    @pl.when(k == nk - 1)
    def _():
        h1 = jnp.maximum(acc_ref[sl, :] + b1_ref[...], 0.0)
        h2 = jnp.dot(h1, w2_ref[...], preferred_element_type=jnp.float32)
        h2 = jnp.maximum(h2 + b2_ref[...], 0.0)
        out_ref[...] = (
            jnp.dot(h2, w3_ref[...], preferred_element_type=jnp.float32)
            + b3_ref[...]
        )


def _tc_mlp(flat3, W1, b1, W2, b2, W3, b3):
    grid = (_K // _KB, _CB // _BB)
    return pl.pallas_call(
        _mlp_body,
        grid=grid,
        in_specs=[
            pl.BlockSpec((_BB, _KB // 128, 128), lambda k, b: (b, k, 0)),
            pl.BlockSpec((_KB, _HID), lambda k, b: (k, 0)),
            pl.BlockSpec((1, _HID), lambda k, b: (0, 0)),
            pl.BlockSpec((_HID, _HID), lambda k, b: (0, 0)),
            pl.BlockSpec((1, _HID), lambda k, b: (0, 0)),
            pl.BlockSpec((_HID, _NA), lambda k, b: (0, 0)),
            pl.BlockSpec((1, _NA), lambda k, b: (0, 0)),
        ],
        out_specs=pl.BlockSpec((_BB, _NA), lambda k, b: (b, 0)),
        out_shape=jax.ShapeDtypeStruct((_CB, _NA), jnp.float32),
        scratch_shapes=[pltpu.VMEM((_CB, _HID), jnp.float32)],
        compiler_params=pltpu.CompilerParams(
            dimension_semantics=("arbitrary", "arbitrary"),
        ),
    )(flat3, W1, b1, W2, b2, W3, b3)


def kernel(board, emb_table, W1, b1, W2, b2, W3, b3):
    board4 = board.astype(jnp.int32).reshape(_NCHUNK, _CB, 8, 128)
    emb_table2 = emb_table.reshape(_N_TILES * _EMBED // 128, 128)
    b1r, b2r, b3r = b1.reshape(1, _HID), b2.reshape(1, _HID), b3.reshape(1, _NA)
    outs = []
    for c in range(_NCHUNK):
        flat3 = _sc_gather(emb_table2, board4[c])
        outs.append(_tc_mlp(flat3, W1, b1r, W2, b2r, W3, b3r))
    return jnp.concatenate(outs, axis=0)


# speedup vs baseline: 1.0773x; 1.0773x over previous
import jax
import jax.numpy as jnp
from jax import lax
from jax.experimental import pallas as pl
from jax.experimental.pallas import tpu as pltpu
from jax.experimental.pallas import tpu_sc as plsc

_N_TILES = 1024
_EMBED = 32
_HID = 256
_NA = 4
_B = 4096

_KP = _N_TILES * _EMBED // 2
_WPR = 16

_NC, _NS = 2, 16
_NW = _NC * _NS
_NCHUNK = 4
_CB = _B // _NCHUNK
_BPW = _CB // _NW


def _sc_gather_body(table_hbm, board_hbm, out_hbm, table_v, idx_v, emb_v,
                    sem_out, sem_idx):
    w = lax.axis_index("s") * _NC + lax.axis_index("c")
    base = w * _BPW
    pltpu.sync_copy(table_hbm, table_v)
    iota = lax.iota(jnp.int32, 16)
    c127 = jnp.full((16,), 127, jnp.int32)

    pltpu.async_copy(board_hbm.at[base], idx_v.at[0], sem_idx).wait()

    def row_body(i, carry):
        br = base + i
        p = lax.rem(i, 2)
        @pl.when(i + 1 < _BPW)
        def _():
            pltpu.async_copy(board_hbm.at[br + 1], idx_v.at[1 - p], sem_idx)
        @pl.when(i >= 2)
        def _():
            pltpu.make_async_copy(emb_v.at[p], out_hbm.at[base], sem_out).wait()

        @plsc.parallel_loop(0, _KP // 128, unroll=4)
        def grp_body(g):
            n0 = g * 8
            rowv = jnp.full((16,), n0 >> 7, jnp.int32)
            col0 = jnp.full((16,), n0 & 127, jnp.int32)
            for c in range(8):
                vsp = plsc.load_gather(idx_v.at[p], [rowv, col0 + c])
                srow = vsp >> 3
                e0 = ((vsp << 4) & c127) + iota
                g0 = plsc.load_gather(table_v, [srow, e0])
                emb_v[p, g, pl.ds(c * 16, 16)] = g0

        pltpu.async_copy(emb_v.at[p], out_hbm.at[br], sem_out)

        @pl.when(i + 1 < _BPW)
        def _():
            pltpu.make_async_copy(board_hbm.at[br], idx_v.at[1 - p],
                                  sem_idx).wait()
        return carry

    lax.fori_loop(0, _BPW, row_body, 0)
    pltpu.make_async_copy(emb_v.at[0], out_hbm.at[base], sem_out).wait()
    pltpu.make_async_copy(emb_v.at[0], out_hbm.at[base], sem_out).wait()


def _sc_gather(table_p, board3):
    mesh = plsc.VectorSubcoreMesh(core_axis_name="c", subcore_axis_name="s")
    f = pl.kernel(
        _sc_gather_body,
        out_type=jax.ShapeDtypeStruct((_CB, _KP // 128, 128), jnp.float32),
        mesh=mesh,
        scratch_types=[
            pltpu.VMEM((_N_TILES * _WPR // 128, 128), jnp.float32),
            pltpu.VMEM((2, 8, 128), jnp.int32),
            pltpu.VMEM((2, _KP // 128, 128), jnp.float32),
            pltpu.SemaphoreType.DMA,
            pltpu.SemaphoreType.DMA,
        ],
        compiler_params=pltpu.CompilerParams(use_tc_tiling_on_sc=True,
                                             needs_layout_passes=False),
    )
    return f(table_p, board3)


_BB = 512
_KBP = 2048


def _mlp_body(flat_ref, w1e_ref, w1o_ref, b1_ref, w2_ref, b2_ref, w3_ref,
              b3_ref, out_ref, acc_ref):
    k = pl.program_id(0)
    b = pl.program_id(1)
    nk = pl.num_programs(0)
    xw = lax.bitcast_convert_type(flat_ref[...].reshape(_BB, _KBP), jnp.int32)
    xlo = lax.bitcast_convert_type(xw << 16, jnp.float32)
    xhi = lax.bitcast_convert_type(xw & jnp.int32(-65536), jnp.float32)
    part = jnp.dot(xlo, w1e_ref[...], preferred_element_type=jnp.float32)
    part = part + jnp.dot(xhi, w1o_ref[...], preferred_element_type=jnp.float32)
    sl = pl.ds(b * _BB, _BB)

    @pl.when(k == 0)
    def _():
        acc_ref[sl, :] = part

    @pl.when(k > 0)
    def _():
        acc_ref[sl, :] = acc_ref[sl, :] + part

    @pl.when(k == nk - 1)
    def _():
        h1 = jnp.maximum(acc_ref[sl, :] + b1_ref[...], 0.0)
        h2 = jnp.dot(h1, w2_ref[...], preferred_element_type=jnp.float32)
        h2 = jnp.maximum(h2 + b2_ref[...], 0.0)
        out_ref[...] = (
            jnp.dot(h2, w3_ref[...], preferred_element_type=jnp.float32)
            + b3_ref[...]
        )


def _tc_mlp(flat3, W1e, W1o, b1, W2, b2, W3, b3):
    grid = (_KP // _KBP, _CB // _BB)
    return pl.pallas_call(
        _mlp_body,
        grid=grid,
        in_specs=[
            pl.BlockSpec((_BB, _KBP // 128, 128), lambda k, b: (b, k, 0)),
            pl.BlockSpec((_KBP, _HID), lambda k, b: (k, 0)),
            pl.BlockSpec((_KBP, _HID), lambda k, b: (k, 0)),
            pl.BlockSpec((1, _HID), lambda k, b: (0, 0)),
            pl.BlockSpec((_HID, _HID), lambda k, b: (0, 0)),
            pl.BlockSpec((1, _HID), lambda k, b: (0, 0)),
            pl.BlockSpec((_HID, _NA), lambda k, b: (0, 0)),
            pl.BlockSpec((1, _NA), lambda k, b: (0, 0)),
        ],
        out_specs=pl.BlockSpec((_BB, _NA), lambda k, b: (b, 0)),
        out_shape=jax.ShapeDtypeStruct((_CB, _NA), jnp.float32),
        scratch_shapes=[pltpu.VMEM((_CB, _HID), jnp.float32)],
        compiler_params=pltpu.CompilerParams(
            dimension_semantics=("arbitrary", "arbitrary"),
        ),
    )(flat3, W1e, W1o, b1, W2, b2, W3, b3)


def kernel(board, emb_table, W1, b1, W2, b2, W3, b3):
    board4 = board.astype(jnp.int32).reshape(_NCHUNK, _CB, 8, 128)
    tb = emb_table.astype(jnp.bfloat16)
    tb_i = lax.bitcast_convert_type(tb, jnp.uint16).astype(jnp.uint32)
    packed = tb_i[:, 0::2] | (tb_i[:, 1::2] << 16)
    table_p = lax.bitcast_convert_type(packed, jnp.float32) \
        .reshape(_N_TILES * _WPR // 128, 128)
    W1e = W1[0::2]
    W1o = W1[1::2]
    b1r, b2r, b3r = b1.reshape(1, _HID), b2.reshape(1, _HID), b3.reshape(1, _NA)
    outs = []
    for c in range(_NCHUNK):
        flat3 = _sc_gather(table_p, board4[c])
        outs.append(_tc_mlp(flat3, W1e, W1o, b1r, W2, b2r, W3, b3r))
    return jnp.concatenate(outs, axis=0)


# speedup vs baseline: 1.4651x; 1.3601x over previous
import jax
import jax.numpy as jnp
from jax import lax
from jax.experimental import pallas as pl
from jax.experimental.pallas import tpu as pltpu
from jax.experimental.pallas import tpu_sc as plsc

_N_TILES = 1024
_EMBED = 32
_HID = 256
_NA = 4
_B = 4096

_KP = _N_TILES * _EMBED // 2
_WPR = 16

_NC, _NS = 2, 16
_NW = _NC * _NS
_NCHUNK = 4
_CB = _B // _NCHUNK
_BPW = _CB // _NW


def _sc_gather_body(table_hbm, board_hbm, out_hbm, table_v, idx_v, emb_v,
                    sem_out, sem_idx):
    w = lax.axis_index("s") * _NC + lax.axis_index("c")
    base = w * _BPW
    pltpu.sync_copy(table_hbm, table_v)
    iota = lax.iota(jnp.int32, 16)
    c127 = jnp.full((16,), 127, jnp.int32)

    pltpu.async_copy(board_hbm.at[base], idx_v.at[0], sem_idx).wait()

    def row_body(i, carry):
        br = base + i
        p = lax.rem(i, 2)
        @pl.when(i + 1 < _BPW)
        def _():
            pltpu.async_copy(board_hbm.at[br + 1], idx_v.at[1 - p], sem_idx)
        @pl.when(i >= 2)
        def _():
            pltpu.make_async_copy(emb_v.at[p], out_hbm.at[base], sem_out).wait()

        @plsc.parallel_loop(0, _KP // 128, unroll=4)
        def grp_body(g):
            n0 = g * 8
            rowv = jnp.full((16,), n0 >> 7, jnp.int32)
            col0 = jnp.full((16,), n0 & 127, jnp.int32)
            for c in range(8):
                vsp = plsc.load_gather(idx_v.at[p], [rowv, col0 + c])
                srow = vsp >> 3
                e0 = ((vsp << 4) & c127) + iota
                g0 = plsc.load_gather(table_v, [srow, e0])
                emb_v[p, g, pl.ds(c * 16, 16)] = g0

        pltpu.async_copy(emb_v.at[p], out_hbm.at[br], sem_out)

        @pl.when(i + 1 < _BPW)
        def _():
            pltpu.make_async_copy(board_hbm.at[br], idx_v.at[1 - p],
                                  sem_idx).wait()
        return carry

    lax.fori_loop(0, _BPW, row_body, 0)
    pltpu.make_async_copy(emb_v.at[0], out_hbm.at[base], sem_out).wait()
    pltpu.make_async_copy(emb_v.at[0], out_hbm.at[base], sem_out).wait()


def _sc_gather(table_p, board3):
    mesh = plsc.VectorSubcoreMesh(core_axis_name="c", subcore_axis_name="s")
    f = pl.kernel(
        _sc_gather_body,
        out_type=jax.ShapeDtypeStruct((_CB, _KP // 128, 128), jnp.float32),
        mesh=mesh,
        scratch_types=[
            pltpu.VMEM((_N_TILES * _WPR // 128, 128), jnp.float32),
            pltpu.VMEM((2, 8, 128), jnp.int32),
            pltpu.VMEM((2, _KP // 128, 128), jnp.float32),
            pltpu.SemaphoreType.DMA,
            pltpu.SemaphoreType.DMA,
        ],
        compiler_params=pltpu.CompilerParams(use_tc_tiling_on_sc=True,
                                             needs_layout_passes=False),
    )
    return f(table_p, board3)


_BB = 512
_KBP = 2048


def _mlp_body(flat_ref, w1e_ref, w1o_ref, b1_ref, w2_ref, b2_ref, w3_ref,
              b3_ref, out_ref, acc_ref):
    k = pl.program_id(0)
    b = pl.program_id(1)
    nk = pl.num_programs(0)
    xw = lax.bitcast_convert_type(flat_ref[...].reshape(_BB, _KBP), jnp.int32)
    xlo = lax.bitcast_convert_type(xw << 16, jnp.float32)
    xhi = lax.bitcast_convert_type(xw & jnp.int32(-65536), jnp.float32)
    w1e = w1e_ref[...].reshape(_KBP, _HID)
    w1o = w1o_ref[...].reshape(_KBP, _HID)
    part = jnp.dot(xlo, w1e, preferred_element_type=jnp.float32)
    part = part + jnp.dot(xhi, w1o, preferred_element_type=jnp.float32)
    sl = pl.ds(b * _BB, _BB)

    @pl.when(k == 0)
    def _():
        acc_ref[sl, :] = part

    @pl.when(k > 0)
    def _():
        acc_ref[sl, :] = acc_ref[sl, :] + part

    @pl.when(k == nk - 1)
    def _():
        h1 = jnp.maximum(acc_ref[sl, :] + b1_ref[...], 0.0)
        h2 = jnp.dot(h1, w2_ref[...], preferred_element_type=jnp.float32)
        h2 = jnp.maximum(h2 + b2_ref[...], 0.0)
        out_ref[...] = (
            jnp.dot(h2, w3_ref[...], preferred_element_type=jnp.float32)
            + b3_ref[...]
        )


def _tc_mlp(flat3, W1e, W1o, b1, W2, b2, W3, b3):
    grid = (_KP // _KBP, _CB // _BB)
    return pl.pallas_call(
        _mlp_body,
        grid=grid,
        in_specs=[
            pl.BlockSpec((_BB, _KBP // 128, 128), lambda k, b: (b, k, 0)),
            pl.BlockSpec((_KBP // 16, 1, 16, _HID), lambda k, b: (k, 0, 0, 0)),
            pl.BlockSpec((_KBP // 16, 1, 16, _HID), lambda k, b: (k, 1, 0, 0)),
            pl.BlockSpec((1, _HID), lambda k, b: (0, 0)),
            pl.BlockSpec((_HID, _HID), lambda k, b: (0, 0)),
            pl.BlockSpec((1, _HID), lambda k, b: (0, 0)),
            pl.BlockSpec((_HID, _NA), lambda k, b: (0, 0)),
            pl.BlockSpec((1, _NA), lambda k, b: (0, 0)),
        ],
        out_specs=pl.BlockSpec((_BB, _NA), lambda k, b: (b, 0)),
        out_shape=jax.ShapeDtypeStruct((_CB, _NA), jnp.float32),
        scratch_shapes=[pltpu.VMEM((_CB, _HID), jnp.float32)],
        compiler_params=pltpu.CompilerParams(
            dimension_semantics=("arbitrary", "arbitrary"),
        ),
    )(flat3, W1e, W1o, b1, W2, b2, W3, b3)


def kernel(board, emb_table, W1, b1, W2, b2, W3, b3):
    board4 = board.astype(jnp.int32).reshape(_NCHUNK, _CB, 8, 128)
    tb = emb_table.astype(jnp.bfloat16)
    tb_i = lax.bitcast_convert_type(tb, jnp.uint16).astype(jnp.uint32)
    packed = tb_i[:, :16] | (tb_i[:, 16:] << 16)
    table_p = lax.bitcast_convert_type(packed, jnp.float32) \
        .reshape(_N_TILES * _WPR // 128, 128)
    W1r = W1.reshape(_N_TILES, 2, 16, _HID)
    b1r, b2r, b3r = b1.reshape(1, _HID), b2.reshape(1, _HID), b3.reshape(1, _NA)
    outs = []
    for c in range(_NCHUNK):
        flat3 = _sc_gather(table_p, board4[c])
        outs.append(_tc_mlp(flat3, W1r, W1r, b1r, W2, b2r, W3, b3r))
    return jnp.concatenate(outs, axis=0)
